# baseline (speedup 1.0000x reference)
import functools

import jax
import jax.numpy as jnp
from jax import lax
from jax.experimental import pallas as pl
from jax.experimental.pallas import tpu as pltpu
from jax.experimental.pallas import tpu_sc as plsc

IDP = 64


@functools.cache
def _make_gather(V, S, D, B):
    info = plsc.get_sparse_core_info()
    NC, NS = info.num_cores, info.num_subcores
    NW = NC * NS
    assert B % NW == 0
    b_per_w = B // NW
    n_ch = b_per_w
    NBUF = 4
    assert n_ch >= 8
    n_main = (n_ch - 6) // NBUF
    n_back = n_ch - 2 - NBUF * n_main
    mesh = plsc.VectorSubcoreMesh(core_axis_name="c", subcore_axis_name="s")

    @functools.partial(
        pl.kernel,
        mesh=mesh,
        out_type=jax.ShapeDtypeStruct((B, S, D), jnp.float32),
        compiler_params=pltpu.CompilerParams(use_tc_tiling_on_sc=True),
        scratch_types=[
            pltpu.VMEM((n_ch * IDP,), jnp.int32),
            pltpu.VMEM((NBUF, S, D), jnp.float32),
        ] + [pltpu.SemaphoreType.DMA] * (2 * NBUF),
    )
    def gather_kernel(ids_hbm, table_hbm, out_hbm, idx_v, bufs, *sems):
        sgs = sems[:NBUF]
        sss = sems[NBUF:]
        wid = lax.axis_index("s") * NC + lax.axis_index("c")
        base = wid * b_per_w
        pltpu.sync_copy(ids_hbm.at[pl.ds(base * IDP, n_ch * IDP)], idx_v)

        def start_gather(j, b):
            pltpu.async_copy(
                table_hbm.at[idx_v.at[pl.ds(j * IDP, S)]], bufs.at[b], sgs[b])

        def wait_gather(b):
            pltpu.make_async_copy(
                table_hbm.at[idx_v.at[pl.ds(0, S)]], bufs.at[b], sgs[b]).wait()

        def start_store(j, b):
            pltpu.async_copy(bufs.at[b], out_hbm.at[base + j], sss[b])

        def wait_store(b):
            pltpu.make_async_copy(
                bufs.at[b], out_hbm.at[base], sss[b]).wait()

        start_gather(0, 0)
        start_gather(1, 1)
        for j in (0, 1):
            start_gather(j + 2, (j + 2) % NBUF)
            wait_gather(j % NBUF)
            start_store(j, j % NBUF)

        def body(g, carry):
            j0 = 2 + NBUF * g
            for i in range(NBUF):
                b = (2 + i) % NBUF
                wait_store(i % NBUF)
                start_gather(j0 + i + 2, i % NBUF)
                wait_gather(b)
                start_store(j0 + i, b)
            return carry

        lax.fori_loop(0, n_main, body, 0)

        for j in range(n_ch - n_back, n_ch):
            b = j % NBUF
            if j + 2 < n_ch:
                wait_store((j + 2) % NBUF)
                start_gather(j + 2, (j + 2) % NBUF)
            wait_gather(b)
            start_store(j, b)
        for j in range(n_ch - 4, n_ch):
            wait_store(j % NBUF)

    return gather_kernel


def kernel(ids, weight):
    B, S = ids.shape
    V, D = weight.shape
    ids_pad = jnp.pad(ids.astype(jnp.int32), ((0, 0), (0, IDP - S)))
    return _make_gather(V, S, D, B)(ids_pad.reshape(B * IDP), weight)


# speedup vs baseline: 1.8018x; 1.8018x over previous
import functools

import jax
import jax.numpy as jnp
from jax import lax
from jax.experimental import pallas as pl
from jax.experimental.pallas import tpu as pltpu
from jax.experimental.pallas import tpu_sc as plsc

BPW = 128


@functools.cache
def _make_gather(V, S, D, B):
    info = plsc.get_sparse_core_info()
    NC, NS = info.num_cores, info.num_subcores
    NW = NC * NS
    assert B == NW * BPW
    n_ch = S
    NBUF = 4
    assert n_ch >= 8
    n_main = (n_ch - 6) // NBUF
    n_back = n_ch - 2 - NBUF * n_main
    mesh = plsc.VectorSubcoreMesh(core_axis_name="c", subcore_axis_name="s")

    @functools.partial(
        pl.kernel,
        mesh=mesh,
        out_type=jax.ShapeDtypeStruct((S, B, D), jnp.float32),
        scratch_types=[
            pltpu.VMEM((S, 1, BPW), jnp.int32),
            pltpu.VMEM((NBUF, BPW, D), jnp.float32),
        ] + [pltpu.SemaphoreType.DMA] * (2 * NBUF),
    )
    def gather_kernel(ids_hbm, table_hbm, out_hbm, idx_v, bufs, *sems):
        sgs = sems[:NBUF]
        sss = sems[NBUF:]
        wid = lax.axis_index("s") * NC + lax.axis_index("c")
        base = wid * BPW
        pltpu.sync_copy(ids_hbm.at[wid], idx_v)

        def start_gather(j, b):
            pltpu.async_copy(table_hbm.at[idx_v.at[j, 0]], bufs.at[b], sgs[b])

        def wait_gather(b):
            pltpu.make_async_copy(
                table_hbm.at[idx_v.at[0, 0]], bufs.at[b], sgs[b]).wait()

        def start_store(j, b):
            pltpu.async_copy(
                bufs.at[b], out_hbm.at[j, pl.ds(base, BPW)], sss[b])

        def wait_store(b):
            pltpu.make_async_copy(
                bufs.at[b], out_hbm.at[0, pl.ds(base, BPW)], sss[b]).wait()

        start_gather(0, 0)
        start_gather(1, 1)
        for j in (0, 1):
            start_gather(j + 2, (j + 2) % NBUF)
            wait_gather(j % NBUF)
            start_store(j, j % NBUF)

        def body(g, carry):
            j0 = 2 + NBUF * g
            for i in range(NBUF):
                b = (2 + i) % NBUF
                wait_store(i % NBUF)
                start_gather(j0 + i + 2, i % NBUF)
                wait_gather(b)
                start_store(j0 + i, b)
            return carry

        lax.fori_loop(0, n_main, body, 0)

        for j in range(n_ch - n_back, n_ch):
            b = j % NBUF
            if j + 2 < n_ch:
                wait_store((j + 2) % NBUF)
                start_gather(j + 2, (j + 2) % NBUF)
            wait_gather(b)
            start_store(j, b)
        for j in range(n_ch - 4, n_ch):
            wait_store(j % NBUF)

    return gather_kernel


def kernel(ids, weight):
    B, S = ids.shape
    V, D = weight.shape
    ids_t = (ids.astype(jnp.int32)
             .reshape(B // BPW, BPW, S)
             .transpose(0, 2, 1)
             .reshape(B // BPW, S, 1, BPW))
    out_sbd = _make_gather(V, S, D, B)(ids_t, weight)
    return out_sbd.transpose(1, 0, 2)


# speedup vs baseline: 1.8230x; 1.0118x over previous
import functools

import jax
import jax.numpy as jnp
from jax import lax
from jax.experimental import pallas as pl
from jax.experimental.pallas import tpu as pltpu
from jax.experimental.pallas import tpu_sc as plsc

BPW = 128


@functools.cache
def _make_gather(V, S, D, B):
    info = plsc.get_sparse_core_info()
    NC, NS = info.num_cores, info.num_subcores
    NW = NC * NS
    assert B == NW * BPW
    n_ch = S
    NBUF = 6
    DEPTH = 3
    assert n_ch >= 2 * NBUF
    n_main = (n_ch - 2 * DEPTH) // NBUF
    n_back = n_ch - DEPTH - NBUF * n_main
    assert n_back >= DEPTH
    mesh = plsc.VectorSubcoreMesh(core_axis_name="c", subcore_axis_name="s")

    @functools.partial(
        pl.kernel,
        mesh=mesh,
        out_type=jax.ShapeDtypeStruct((S, B, D), jnp.float32),
        scratch_types=[
            pltpu.VMEM((S, 1, BPW), jnp.int32),
            pltpu.VMEM((NBUF, BPW, D), jnp.float32),
        ] + [pltpu.SemaphoreType.DMA] * (2 * NBUF),
    )
    def gather_kernel(ids_hbm, table_hbm, out_hbm, idx_v, bufs, *sems):
        sgs = sems[:NBUF]
        sss = sems[NBUF:]
        wid = lax.axis_index("s") * NC + lax.axis_index("c")
        base = wid * BPW
        pltpu.sync_copy(ids_hbm.at[wid], idx_v)

        def start_gather(j, b):
            pltpu.async_copy(table_hbm.at[idx_v.at[j, 0]], bufs.at[b], sgs[b])

        def wait_gather(b):
            pltpu.make_async_copy(
                table_hbm.at[idx_v.at[0, 0]], bufs.at[b], sgs[b]).wait()

        def start_store(j, b):
            pltpu.async_copy(
                bufs.at[b], out_hbm.at[j, pl.ds(base, BPW)], sss[b])

        def wait_store(b):
            pltpu.make_async_copy(
                bufs.at[b], out_hbm.at[0, pl.ds(base, BPW)], sss[b]).wait()

        for j in range(DEPTH):
            start_gather(j, j % NBUF)
        for j in range(DEPTH):
            start_gather(j + DEPTH, (j + DEPTH) % NBUF)
            wait_gather(j % NBUF)
            start_store(j, j % NBUF)

        def body(g, carry):
            j0 = DEPTH + NBUF * g
            for i in range(NBUF):
                b = (DEPTH + i) % NBUF
                wait_store((DEPTH + i + DEPTH) % NBUF)
                start_gather(j0 + i + DEPTH, (DEPTH + i + DEPTH) % NBUF)
                wait_gather(b)
                start_store(j0 + i, b)
            return carry

        lax.fori_loop(0, n_main, body, 0)

        for j in range(n_ch - n_back, n_ch):
            b = j % NBUF
            if j + DEPTH < n_ch:
                wait_store((j + DEPTH) % NBUF)
                start_gather(j + DEPTH, (j + DEPTH) % NBUF)
            wait_gather(b)
            start_store(j, b)
        for j in range(n_ch - NBUF, n_ch):
            wait_store(j % NBUF)

    return gather_kernel


def kernel(ids, weight):
    B, S = ids.shape
    V, D = weight.shape
    ids_t = (ids.astype(jnp.int32)
             .reshape(B // BPW, BPW, S)
             .transpose(0, 2, 1)
             .reshape(B // BPW, S, 1, BPW))
    out_sbd = _make_gather(V, S, D, B)(ids_t, weight)
    return out_sbd.transpose(1, 0, 2)
